# baseline (speedup 1.0000x reference)
import functools

import jax
import jax.numpy as jnp
from jax import lax
from jax.experimental import pallas as pl
from jax.experimental.pallas import tpu as pltpu
from jax.experimental.pallas import tpu_sc as plsc

N = 10000
E = 320000
D = 128
H = 32
HH = 2 * H
NC, NS = 2, 16
NW = NC * NS
CH = 128
CPW = 78
TAIL_W = (E - NW * CPW * CH) // CH
TAIL_BASE = NW * CPW * CH
NP2 = 10240
EPS = 1e-5
T2R = 4000
NB1 = 3
NB3 = 3

_SC_PARAMS = pltpu.CompilerParams(use_tc_tiling_on_sc=False,
                                  needs_layout_passes=False)


def _blkdiag(A, B):
    za = jnp.zeros((A.shape[0], B.shape[1]), A.dtype)
    zb = jnp.zeros((B.shape[0], A.shape[1]), B.dtype)
    return jnp.concatenate(
        [jnp.concatenate([A, za], axis=1), jnp.concatenate([zb, B], axis=1)], axis=0)


def _node_prep_body(x_ref, f1w_ref, f1b_ref, f1g_ref, f1be_ref,
                    f2w_ref, f2b_ref, f2g_ref, f2be_ref,
                    l0w_ref, l0b_ref, pw_ref, qw_ref,
                    a_ref, bq_ref, z0_ref):
    def bn_relu(h, g, be):
        m = jnp.mean(h, axis=0, keepdims=True)
        v = jnp.mean((h - m) * (h - m), axis=0, keepdims=True)
        return jnp.maximum((h - m) * lax.rsqrt(v + EPS) * g + be, 0.0)

    h = jnp.dot(x_ref[...], f1w_ref[...], preferred_element_type=jnp.float32)
    h = bn_relu(h + f1b_ref[...], f1g_ref[...], f1be_ref[...])
    h = jnp.dot(h, f2w_ref[...], preferred_element_type=jnp.float32)
    h = bn_relu(h + f2b_ref[...], f2g_ref[...], f2be_ref[...])
    z0_ref[...] = (jnp.dot(h, l0w_ref[...], preferred_element_type=jnp.float32)
                   + l0b_ref[...])
    a_ref[...] = jnp.dot(h, pw_ref[...], preferred_element_type=jnp.float32)
    bq_ref[...] = jnp.dot(h, qw_ref[...], preferred_element_type=jnp.float32)


def _node_prep(x, f1w, f1b, f1g, f1be, f2w, f2b, f2g, f2be, l0w, l0b, pw, qw):
    return pl.pallas_call(
        _node_prep_body,
        out_shape=[
            jax.ShapeDtypeStruct((N, HH), jnp.float32),
            jax.ShapeDtypeStruct((N, HH), jnp.float32),
            jax.ShapeDtypeStruct((N, 2), jnp.float32),
        ],
    )(x, f1w, f1b, f1g, f1be, f2w, f2b, f2g, f2be, l0w, l0b, pw, qw)


def _sc_pass1_body(a_hbm, bq_hbm, dst_hbm, src_hbm,
                   u_hbm, stats_hbm, cnt_hbm,
                   dstall, srcall, ars, brs, ubs, hist, accv,
                   sgas, sgbs, sos):
    c = lax.axis_index("c")
    s = lax.axis_index("s")
    wid = c * NS + s
    base = wid * (CPW * CH)
    npre = CPW * CH
    ones16 = jnp.ones((16,), jnp.float32)
    zero16 = jnp.zeros((16,), jnp.float32)

    pltpu.sync_copy(dst_hbm.at[pl.ds(base, npre)], dstall.at[pl.ds(0, npre)])
    pltpu.sync_copy(src_hbm.at[pl.ds(base, npre)], srcall.at[pl.ds(0, npre)])

    @pl.when(wid < TAIL_W)
    def _():
        tb = TAIL_BASE + wid * CH
        pltpu.sync_copy(dst_hbm.at[pl.ds(tb, CH)], dstall.at[pl.ds(npre, CH)])
        pltpu.sync_copy(src_hbm.at[pl.ds(tb, CH)], srcall.at[pl.ds(npre, CH)])

    def zb(j, _):
        hist[pl.ds(j * 16, 16)] = zero16
        return 0
    lax.fori_loop(0, NP2 // 16, zb, 0)

    def fire(g, b):
        pltpu.async_copy(a_hbm.at[dstall.at[pl.ds(g * CH, CH)]], ars.at[b], sgas[b])
        pltpu.async_copy(bq_hbm.at[srcall.at[pl.ds(g * CH, CH)]], brs.at[b], sgbs[b])

    def wait_gather(b):
        pltpu.make_async_copy(a_hbm.at[dstall.at[pl.ds(0, CH)]], ars.at[b], sgas[b]).wait()
        pltpu.make_async_copy(bq_hbm.at[srcall.at[pl.ds(0, CH)]], brs.at[b], sgbs[b]).wait()

    def wait_out(b):
        pltpu.make_async_copy(ubs.at[b], u_hbm.at[pl.ds(base // 2, CH // 2)],
                              sos[b]).wait()

    def compute(g, b, accs):
        ar, br, ub = ars.at[b], brs.at[b], ubs.at[b]

        def row_body(t, acc):
            s0, s1, s2, s3, q0, q1, q2, q3 = acc
            r0 = 2 * t
            r1 = 2 * t + 1
            a0 = ar[r0, pl.ds(0, 16)] + br[r0, pl.ds(0, 16)]
            a1 = ar[r0, pl.ds(16, 16)] + br[r0, pl.ds(16, 16)]
            a2 = ar[r0, pl.ds(32, 16)] + br[r0, pl.ds(32, 16)]
            a3 = ar[r0, pl.ds(48, 16)] + br[r0, pl.ds(48, 16)]
            b0 = ar[r1, pl.ds(0, 16)] + br[r1, pl.ds(0, 16)]
            b1 = ar[r1, pl.ds(16, 16)] + br[r1, pl.ds(16, 16)]
            b2 = ar[r1, pl.ds(32, 16)] + br[r1, pl.ds(32, 16)]
            b3 = ar[r1, pl.ds(48, 16)] + br[r1, pl.ds(48, 16)]
            ub[t, pl.ds(0, 32)] = plsc.pack(a0, a1, format=plsc.PackFormat.INTERLEAVED)
            ub[t, pl.ds(32, 32)] = plsc.pack(a2, a3, format=plsc.PackFormat.INTERLEAVED)
            ub[t, pl.ds(64, 32)] = plsc.pack(b0, b1, format=plsc.PackFormat.INTERLEAVED)
            ub[t, pl.ds(96, 32)] = plsc.pack(b2, b3, format=plsc.PackFormat.INTERLEAVED)
            return (s0 + a0 + b0, s1 + a1 + b1, s2 + a2 + b2, s3 + a3 + b3,
                    q0 + a0 * a0 + b0 * b0, q1 + a1 * a1 + b1 * b1,
                    q2 + a2 * a2 + b2 * b2, q3 + a3 * a3 + b3 * b3)

        accs = lax.fori_loop(0, CH // 2, row_body, accs)
        pltpu.async_copy(ub, u_hbm.at[pl.ds((base + g * CH) // 2, CH // 2)],
                         sos[b])
        for j in range(CH // 16):
            iv = dstall[pl.ds(g * CH + j * 16, 16)]
            plsc.addupdate_scatter(hist, [iv], ones16)
        return accs

    accs0 = (zero16,) * 8
    for b in range(NB1 - 1):
        fire(b, b)

    def ring_body(j, accs):
        for b in range(NB1):
            g = j * NB1 + b
            nxt = g + NB1 - 1

            @pl.when(nxt < CPW)
            def _():
                fire(nxt, (b + NB1 - 1) % NB1)

            wait_gather(b)

            @pl.when(j > 0)
            def _():
                wait_out(b)

            accs = compute(g, b, accs)
        return accs

    accs = lax.fori_loop(0, CPW // NB1, ring_body, accs0)

    for b in range(NB1):
        wait_out(b)
    for k in range(8):
        accv[pl.ds(k * 16, 16)] = accs[k]

    @pl.when(wid < TAIL_W)
    def _():
        tb = TAIL_BASE + wid * CH
        ar, br, ub = ars.at[0], brs.at[0], ubs.at[0]
        cp1 = pltpu.async_copy(a_hbm.at[dstall.at[pl.ds(npre, CH)]], ar, sgas[0])
        cp2 = pltpu.async_copy(bq_hbm.at[srcall.at[pl.ds(npre, CH)]], br, sgbs[0])
        cp1.wait()
        cp2.wait()

        def row_body(t, _):
            for half in range(2):
                r = 2 * t + half
                uvs = []
                for k in range(4):
                    sl = pl.ds(k * 16, 16)
                    uv = ar[r, sl] + br[r, sl]
                    uvs.append(uv)
                    sa = pl.ds(k * 16, 16)
                    accv[sa] = accv[sa] + uv
                    sb = pl.ds(64 + k * 16, 16)
                    accv[sb] = accv[sb] + uv * uv
                ub[t, pl.ds(64 * half, 32)] = plsc.pack(
                    uvs[0], uvs[1], format=plsc.PackFormat.INTERLEAVED)
                ub[t, pl.ds(64 * half + 32, 32)] = plsc.pack(
                    uvs[2], uvs[3], format=plsc.PackFormat.INTERLEAVED)
            return 0

        lax.fori_loop(0, CH // 2, row_body, 0)
        pltpu.sync_copy(ub, u_hbm.at[pl.ds(tb // 2, CH // 2)])
        for j in range(CH // 16):
            iv = dstall[pl.ds(npre + j * 16, 16)]
            plsc.addupdate_scatter(hist, [iv], ones16)

    pltpu.sync_copy(accv, stats_hbm.at[wid, 0])
    pltpu.sync_copy(hist, cnt_hbm.at[pl.ds(wid * NP2, NP2)])


def _sc_pass1(a, bq, dst, src):
    mesh = plsc.VectorSubcoreMesh(core_axis_name="c", subcore_axis_name="s")
    f = pl.kernel(
        _sc_pass1_body,
        out_type=[
            jax.ShapeDtypeStruct((E // 2, 2 * HH), jnp.bfloat16),
            jax.ShapeDtypeStruct((NW, 8, 128), jnp.float32),
            jax.ShapeDtypeStruct((NW * NP2,), jnp.float32),
        ],
        mesh=mesh,
        compiler_params=_SC_PARAMS,
        scratch_types=[
            pltpu.VMEM((CPW * CH + CH,), jnp.int32),
            pltpu.VMEM((CPW * CH + CH,), jnp.int32),
            pltpu.VMEM((NB1, CH, HH), jnp.float32),
            pltpu.VMEM((NB1, CH, HH), jnp.float32),
            pltpu.VMEM((NB1, CH // 2, 2 * HH), jnp.bfloat16),
            pltpu.VMEM((NP2,), jnp.float32),
            pltpu.VMEM((128,), jnp.float32),
            [pltpu.SemaphoreType.DMA] * NB1,
            [pltpu.SemaphoreType.DMA] * NB1,
            [pltpu.SemaphoreType.DMA] * NB1,
        ],
    )
    return f(a, bq, dst, src)


def _tc_pass2_body(u_ref, stats_ref, e1g_ref, e1be_ref, e2w_ref, w2p_ref,
                   pm_ref, pmt_ref, e2b_ref, e2g_ref, e2be_ref,
                   v_ref, st_ref, g_scr, sh_scr):
    i = pl.program_id(0)

    @pl.when(i == 0)
    def _():
        g_scr[...] = jnp.zeros((2 * HH, 2 * HH), jnp.float32)
        sh_scr[...] = jnp.zeros((1, 2 * HH), jnp.float32)

    parts = stats_ref[:, 0, :]
    su = jnp.sum(parts[:, :HH], axis=0, keepdims=True)
    sq = jnp.sum(parts[:, HH:], axis=0, keepdims=True)
    mu = su * (1.0 / E)
    var = sq * (1.0 / E) - mu * mu
    s1 = e1g_ref[...] * lax.rsqrt(var + EPS)
    t1 = e1be_ref[...] - mu * s1
    s1w = jnp.concatenate([s1, s1], axis=1)
    t1w = jnp.concatenate([t1, t1], axis=1)
    pm = pm_ref[...]
    s1p = jnp.dot(s1w, pm, preferred_element_type=jnp.float32)
    t1p = jnp.dot(t1w, pm, preferred_element_type=jnp.float32)

    uf = u_ref[...].astype(jnp.float32)
    h1p = jnp.maximum(uf * s1p + t1p, 0.0)
    sh_scr[...] += jnp.sum(h1p, axis=0, keepdims=True)
    g_scr[...] += lax.dot_general(h1p, h1p, (((0,), (0,)), ((), ())),
                                  preferred_element_type=jnp.float32)
    e2b = e2b_ref[...]
    e2bw = jnp.concatenate([e2b, e2b], axis=1)
    v_ref[...] = (jnp.dot(h1p, w2p_ref[...], preferred_element_type=jnp.float32)
                  + e2bw)

    @pl.when(i == E // 2 // T2R - 1)
    def _():
        pmt = pmt_ref[...]
        w2 = e2w_ref[...]
        sh_true = jnp.dot(sh_scr[...], pmt, preferred_element_type=jnp.float32)
        gt = jnp.dot(pm, jnp.dot(g_scr[...], pmt,
                                 preferred_element_type=jnp.float32),
                     preferred_element_type=jnp.float32)
        sfold = sh_true[:, :HH] + sh_true[:, HH:]
        esh = sfold * (1.0 / E)
        meanv = jnp.dot(esh, w2, preferred_element_type=jnp.float32) + e2b
        gf = gt[:HH, :HH] + gt[HH:, HH:]
        gw = jnp.dot(gf, w2, preferred_element_type=jnp.float32)
        diag = jnp.sum(w2 * gw, axis=0, keepdims=True) * (1.0 / E)
        ev2 = diag + 2.0 * e2b * (meanv - e2b) + e2b * e2b
        var2 = ev2 - meanv * meanv
        s2 = e2g_ref[...] * lax.rsqrt(var2 + EPS)
        t2 = e2be_ref[...] - meanv * s2
        st_ref[...] = jnp.concatenate([s2, t2], axis=0)


def _tc_pass2(u, stats, e1g, e1be, e2w, w2p, pm, pmt, e2b, e2g, e2be):
    grid = (E // 2 // T2R,)
    small = lambda shp: pl.BlockSpec(shp, lambda i: (0, 0))
    return pl.pallas_call(
        _tc_pass2_body,
        grid=grid,
        in_specs=[
            pl.BlockSpec((T2R, 2 * HH), lambda i: (i, 0)),
            pl.BlockSpec((NW, 8, 128), lambda i: (0, 0, 0)),
            small((1, HH)), small((1, HH)),
            small((HH, HH)), small((2 * HH, 2 * HH)),
            small((2 * HH, 2 * HH)), small((2 * HH, 2 * HH)),
            small((1, HH)), small((1, HH)), small((1, HH)),
        ],
        out_specs=[
            pl.BlockSpec((T2R, 2 * HH), lambda i: (i, 0)),
            pl.BlockSpec((2, HH), lambda i: (0, 0)),
        ],
        out_shape=[
            jax.ShapeDtypeStruct((E // 2, 2 * HH), jnp.float32),
            jax.ShapeDtypeStruct((2, HH), jnp.float32),
        ],
        scratch_shapes=[
            pltpu.VMEM((2 * HH, 2 * HH), jnp.float32),
            pltpu.VMEM((1, 2 * HH), jnp.float32),
        ],
    )(u, stats, e1g, e1be, e2w, w2p, pm, pmt, e2b, e2g, e2be)


def _sc_pass3_body(v_hbm, dst_hbm, st_hbm, za_hbm,
                   agg_hbm,
                   dst2d, vrs, mbs, stv, agg_sp,
                   isem, svs, sss):
    c = lax.axis_index("c")
    s = lax.axis_index("s")
    wid = c * NS + s
    base = wid * (CPW * CH)
    rpt = NP2 // NS
    pltpu.sync_copy(za_hbm.at[pl.ds(s * rpt, rpt)], agg_sp.at[pl.ds(s * rpt, rpt)])
    pltpu.sync_copy(st_hbm, stv)

    descs = []
    for j in range(CPW):
        descs.append(pltpu.async_copy(
            dst_hbm.at[pl.ds(base + j * CH, CH)], dst2d.at[j], isem))
    for d in descs:
        d.wait()

    @pl.when(wid < TAIL_W)
    def _():
        pltpu.sync_copy(dst_hbm.at[pl.ds(TAIL_BASE + wid * CH, CH)],
                        dst2d.at[CPW])

    plsc.subcore_barrier()

    def fire_v(g, b):
        pltpu.async_copy(v_hbm.at[pl.ds((base + g * CH) // 2, CH // 2)],
                         vrs.at[b], svs[b])

    def wait_v(b):
        pltpu.make_async_copy(v_hbm.at[pl.ds(base // 2, CH // 2)], vrs.at[b],
                              svs[b]).wait()

    def wait_s(b):
        pltpu.make_async_copy(mbs.at[b], agg_sp.at[dst2d.at[0]], sss[b]).wait()

    def compute(g, b):
        vr, mb = vrs.at[b], mbs.at[b]

        def row_body(t, _):
            for half in range(2):
                for k in range(4):
                    sl = pl.ds(k * 16, 16)
                    sv = pl.ds(64 * half + k * 16, 16)
                    mb[2 * t + half, sl] = jnp.maximum(
                        vr[t, sv] * stv[0, sl] + stv[1, sl], 0.0)
            return 0

        lax.fori_loop(0, CH // 2, row_body, 0)
        pltpu.async_copy(mb, agg_sp.at[dst2d.at[g]], sss[b], add=True)

    for b in range(NB3 - 1):
        fire_v(b, b)

    def ring_body(j, _):
        for b in range(NB3):
            g = j * NB3 + b
            nxt = g + NB3 - 1

            @pl.when(nxt < CPW)
            def _():
                fire_v(nxt, (b + NB3 - 1) % NB3)

            wait_v(b)

            @pl.when(j > 0)
            def _():
                wait_s(b)

            compute(g, b)
        return 0

    lax.fori_loop(0, CPW // NB3, ring_body, 0)
    for b in range(NB3):
        wait_s(b)

    @pl.when(wid < TAIL_W)
    def _():
        tb = TAIL_BASE + wid * CH
        vr, mb = vrs.at[0], mbs.at[0]
        pltpu.sync_copy(v_hbm.at[pl.ds(tb // 2, CH // 2)], vr)

        def row_body(t, _):
            for half in range(2):
                for k in range(4):
                    sl = pl.ds(k * 16, 16)
                    sv = pl.ds(64 * half + k * 16, 16)
                    mb[2 * t + half, sl] = jnp.maximum(
                        vr[t, sv] * stv[0, sl] + stv[1, sl], 0.0)
            return 0

        lax.fori_loop(0, CH // 2, row_body, 0)
        pltpu.sync_copy(mb, agg_sp.at[dst2d.at[CPW]], add=True)

    plsc.subcore_barrier()

    @pl.when(s == 0)
    def _():
        pltpu.sync_copy(agg_sp, agg_hbm.at[c])


def _sc_pass3(v, dst, st):
    za = jnp.zeros((NP2, HH), jnp.float32)
    mesh = plsc.VectorSubcoreMesh(core_axis_name="c", subcore_axis_name="s")
    f = pl.kernel(
        _sc_pass3_body,
        out_type=jax.ShapeDtypeStruct((NC, NP2, HH), jnp.float32),
        mesh=mesh,
        compiler_params=_SC_PARAMS,
        scratch_types=[
            pltpu.VMEM((CPW + 1, CH), jnp.int32),
            pltpu.VMEM((NB3, CH // 2, 2 * HH), jnp.float32),
            pltpu.VMEM((NB3, CH, HH), jnp.float32),
            pltpu.VMEM((2, HH), jnp.float32),
            pltpu.VMEM_SHARED((NP2, HH), jnp.float32),
            pltpu.SemaphoreType.DMA,
            [pltpu.SemaphoreType.DMA] * NB3,
            [pltpu.SemaphoreType.DMA] * NB3,
        ],
    )
    return f(v, dst, st, za)


def _epi_body(agg_ref, cnt_ref, z0_ref, l1w_ref, l1b_ref, out_ref):
    agg = agg_ref[0, :N] + agg_ref[1, :N]
    cnt = jnp.zeros((1, NP2), jnp.float32)
    for w in range(NW):
        cnt = cnt + cnt_ref[:, pl.ds(w * NP2, NP2)]
    cntc = cnt[:, :N].reshape(N, 1)
    h2 = agg / jnp.maximum(cntc, 1.0)
    out_ref[...] = (z0_ref[...]
                    + jnp.dot(h2, l1w_ref[...], preferred_element_type=jnp.float32)
                    + l1b_ref[...])


def _epilogue(agg2, cnt_flat, z0, l1w, l1b):
    return pl.pallas_call(
        _epi_body,
        out_shape=jax.ShapeDtypeStruct((N, 2), jnp.float32),
    )(agg2, cnt_flat.reshape(1, NW * NP2), z0, l1w, l1b)


def kernel(x, paramsE, paramsS, edge_index):
    pE, pS = paramsE, paramsS
    cat = lambda k: jnp.concatenate([pE[k], pS[k]])[None]
    f1w = jnp.concatenate([pE['f1_w'], pS['f1_w']], axis=1)
    f2w = _blkdiag(pE['f2_w'], pS['f2_w'])
    l0w = _blkdiag(pE['l0_w'], pS['l0_w'])
    l1w = _blkdiag(pE['l1_w'], pS['l1_w'])
    pw = _blkdiag(pE['e1_w'][:H] - pE['e1_w'][H:], pS['e1_w'][:H] - pS['e1_w'][H:])
    qw = _blkdiag(pE['e1_w'][H:], pS['e1_w'][H:])
    e2w = _blkdiag(pE['e2_w'], pS['e2_w'])

    a, bq, z0 = _node_prep(
        x, f1w, cat('f1_b'), cat('f1_g'), cat('f1_be'),
        f2w, cat('f2_b'), cat('f2_g'), cat('f2_be'),
        l0w, cat('l0_b'), pw, qw)

    src = edge_index[0].astype(jnp.int32)
    dst = edge_index[1].astype(jnp.int32)

    u, stats, cnt_flat = _sc_pass1(a, bq, dst, src)
    e2w2 = _blkdiag(e2w, e2w)
    perm64 = [0] * 64
    for blk in range(2):
        for i in range(16):
            perm64[32 * blk + 2 * i] = 32 * blk + i
            perm64[32 * blk + 2 * i + 1] = 32 * blk + 16 + i
    permvec = perm64 + [64 + p for p in perm64]
    pmat = jnp.zeros((2 * H * 2, 2 * H * 2), jnp.float32)
    pmat = pmat.at[jnp.asarray(permvec), jnp.arange(128)].set(1.0)
    pmt = pmat.T
    w2p = pmt @ e2w2
    v, st = _tc_pass2(u, stats, cat('e1_g'), cat('e1_be'),
                      e2w, w2p, pmat, pmt, cat('e2_b'), cat('e2_g'),
                      cat('e2_be'))
    agg2 = _sc_pass3(v, dst, st)
    return _epilogue(agg2, cnt_flat, z0, l1w, cat('l1_b'))


# speedup vs baseline: 1.3094x; 1.3094x over previous
import functools

import jax
import jax.numpy as jnp
from jax import lax
from jax.experimental import pallas as pl
from jax.experimental.pallas import tpu as pltpu
from jax.experimental.pallas import tpu_sc as plsc

N = 10000
E = 320000
D = 128
H = 32
HH = 2 * H
NC, NS = 2, 16
NW = NC * NS
CH = 128
CPW = 78
TAIL_W = (E - NW * CPW * CH) // CH
TAIL_BASE = NW * CPW * CH
NP2 = 10240
EPS = 1e-5
T2 = 8000
NB1 = 3
NB3 = 3

_SC_PARAMS = pltpu.CompilerParams(use_tc_tiling_on_sc=False,
                                  needs_layout_passes=False)


def _blkdiag(A, B):
    za = jnp.zeros((A.shape[0], B.shape[1]), A.dtype)
    zb = jnp.zeros((B.shape[0], A.shape[1]), B.dtype)
    return jnp.concatenate(
        [jnp.concatenate([A, za], axis=1), jnp.concatenate([zb, B], axis=1)], axis=0)


def _node_prep_body(x_ref, f1w_ref, f1b_ref, f1g_ref, f1be_ref,
                    f2w_ref, f2b_ref, f2g_ref, f2be_ref,
                    l0w_ref, l0b_ref, pw_ref, qw_ref,
                    a_ref, bq_ref, z0_ref):
    def bn_relu(h, g, be):
        m = jnp.mean(h, axis=0, keepdims=True)
        v = jnp.mean((h - m) * (h - m), axis=0, keepdims=True)
        return jnp.maximum((h - m) * lax.rsqrt(v + EPS) * g + be, 0.0)

    h = jnp.dot(x_ref[...], f1w_ref[...], preferred_element_type=jnp.float32)
    h = bn_relu(h + f1b_ref[...], f1g_ref[...], f1be_ref[...])
    h = jnp.dot(h, f2w_ref[...], preferred_element_type=jnp.float32)
    h = bn_relu(h + f2b_ref[...], f2g_ref[...], f2be_ref[...])
    z0_ref[...] = (jnp.dot(h, l0w_ref[...], preferred_element_type=jnp.float32)
                   + l0b_ref[...])
    a_ref[...] = jnp.dot(h, pw_ref[...], preferred_element_type=jnp.float32)
    bq_ref[...] = jnp.dot(h, qw_ref[...], preferred_element_type=jnp.float32)


def _node_prep(x, f1w, f1b, f1g, f1be, f2w, f2b, f2g, f2be, l0w, l0b, pw, qw):
    return pl.pallas_call(
        _node_prep_body,
        out_shape=[
            jax.ShapeDtypeStruct((N, HH), jnp.float32),
            jax.ShapeDtypeStruct((N, HH), jnp.float32),
            jax.ShapeDtypeStruct((N, 2), jnp.float32),
        ],
    )(x, f1w, f1b, f1g, f1be, f2w, f2b, f2g, f2be, l0w, l0b, pw, qw)


def _sc_pass1_body(a_hbm, bq_hbm, dst_hbm, src_hbm,
                   u_hbm, stats_hbm, cnt_hbm,
                   dstall, srcall, ars, brs, ubs, hist, accv,
                   sgas, sgbs, sos):
    c = lax.axis_index("c")
    s = lax.axis_index("s")
    wid = c * NS + s
    base = wid * (CPW * CH)
    npre = CPW * CH
    ones16 = jnp.ones((16,), jnp.float32)
    zero16 = jnp.zeros((16,), jnp.float32)

    pltpu.sync_copy(dst_hbm.at[pl.ds(base, npre)], dstall.at[pl.ds(0, npre)])
    pltpu.sync_copy(src_hbm.at[pl.ds(base, npre)], srcall.at[pl.ds(0, npre)])

    @pl.when(wid < TAIL_W)
    def _():
        tb = TAIL_BASE + wid * CH
        pltpu.sync_copy(dst_hbm.at[pl.ds(tb, CH)], dstall.at[pl.ds(npre, CH)])
        pltpu.sync_copy(src_hbm.at[pl.ds(tb, CH)], srcall.at[pl.ds(npre, CH)])

    def zb(j, _):
        hist[pl.ds(j * 16, 16)] = zero16
        return 0
    lax.fori_loop(0, NP2 // 16, zb, 0)

    def fire(g, b):
        pltpu.async_copy(a_hbm.at[dstall.at[pl.ds(g * CH, CH)]], ars.at[b], sgas[b])
        pltpu.async_copy(bq_hbm.at[srcall.at[pl.ds(g * CH, CH)]], brs.at[b], sgbs[b])

    def wait_gather(b):
        pltpu.make_async_copy(a_hbm.at[dstall.at[pl.ds(0, CH)]], ars.at[b], sgas[b]).wait()
        pltpu.make_async_copy(bq_hbm.at[srcall.at[pl.ds(0, CH)]], brs.at[b], sgbs[b]).wait()

    def wait_out(b):
        pltpu.make_async_copy(ubs.at[b], u_hbm.at[pl.ds(base // 2, CH // 2)],
                              sos[b]).wait()

    def compute(g, b, accs):
        ar, br, ub = ars.at[b], brs.at[b], ubs.at[b]

        def row_body(t, acc):
            s0, s1, s2, s3, q0, q1, q2, q3 = acc
            r0 = 2 * t
            r1 = 2 * t + 1
            a0 = ar[r0, pl.ds(0, 16)] + br[r0, pl.ds(0, 16)]
            a1 = ar[r0, pl.ds(16, 16)] + br[r0, pl.ds(16, 16)]
            a2 = ar[r0, pl.ds(32, 16)] + br[r0, pl.ds(32, 16)]
            a3 = ar[r0, pl.ds(48, 16)] + br[r0, pl.ds(48, 16)]
            b0 = ar[r1, pl.ds(0, 16)] + br[r1, pl.ds(0, 16)]
            b1 = ar[r1, pl.ds(16, 16)] + br[r1, pl.ds(16, 16)]
            b2 = ar[r1, pl.ds(32, 16)] + br[r1, pl.ds(32, 16)]
            b3 = ar[r1, pl.ds(48, 16)] + br[r1, pl.ds(48, 16)]
            ub[t, pl.ds(0, 16)] = a0
            ub[t, pl.ds(16, 16)] = a1
            ub[t, pl.ds(32, 16)] = a2
            ub[t, pl.ds(48, 16)] = a3
            ub[t, pl.ds(64, 16)] = b0
            ub[t, pl.ds(80, 16)] = b1
            ub[t, pl.ds(96, 16)] = b2
            ub[t, pl.ds(112, 16)] = b3
            return (s0 + a0 + b0, s1 + a1 + b1, s2 + a2 + b2, s3 + a3 + b3,
                    q0 + a0 * a0 + b0 * b0, q1 + a1 * a1 + b1 * b1,
                    q2 + a2 * a2 + b2 * b2, q3 + a3 * a3 + b3 * b3)

        accs = lax.fori_loop(0, CH // 2, row_body, accs)
        pltpu.async_copy(ub, u_hbm.at[pl.ds((base + g * CH) // 2, CH // 2)],
                         sos[b])
        for j in range(CH // 16):
            iv = dstall[pl.ds(g * CH + j * 16, 16)]
            plsc.addupdate_scatter(hist, [iv], ones16)
        return accs

    accs0 = (zero16,) * 8
    for b in range(NB1 - 1):
        fire(b, b)

    def ring_body(j, accs):
        for b in range(NB1):
            g = j * NB1 + b
            nxt = g + NB1 - 1

            @pl.when(nxt < CPW)
            def _():
                fire(nxt, (b + NB1 - 1) % NB1)

            wait_gather(b)

            @pl.when(j > 0)
            def _():
                wait_out(b)

            accs = compute(g, b, accs)
        return accs

    accs = lax.fori_loop(0, CPW // NB1, ring_body, accs0)

    for b in range(NB1):
        wait_out(b)
    for k in range(8):
        accv[pl.ds(k * 16, 16)] = accs[k]

    @pl.when(wid < TAIL_W)
    def _():
        tb = TAIL_BASE + wid * CH
        ar, br, ub = ars.at[0], brs.at[0], ubs.at[0]
        cp1 = pltpu.async_copy(a_hbm.at[dstall.at[pl.ds(npre, CH)]], ar, sgas[0])
        cp2 = pltpu.async_copy(bq_hbm.at[srcall.at[pl.ds(npre, CH)]], br, sgbs[0])
        cp1.wait()
        cp2.wait()

        def row_body(t, _):
            for half in range(2):
                r = 2 * t + half
                for k in range(4):
                    sl = pl.ds(k * 16, 16)
                    uv = ar[r, sl] + br[r, sl]
                    ub[t, pl.ds(64 * half + k * 16, 16)] = uv
                    sa = pl.ds(k * 16, 16)
                    accv[sa] = accv[sa] + uv
                    sb = pl.ds(64 + k * 16, 16)
                    accv[sb] = accv[sb] + uv * uv
            return 0

        lax.fori_loop(0, CH // 2, row_body, 0)
        pltpu.sync_copy(ub, u_hbm.at[pl.ds(tb // 2, CH // 2)])
        for j in range(CH // 16):
            iv = dstall[pl.ds(npre + j * 16, 16)]
            plsc.addupdate_scatter(hist, [iv], ones16)

    pltpu.sync_copy(accv, stats_hbm.at[wid, 0])
    pltpu.sync_copy(hist, cnt_hbm.at[pl.ds(wid * NP2, NP2)])


def _sc_pass1(a, bq, dst, src):
    mesh = plsc.VectorSubcoreMesh(core_axis_name="c", subcore_axis_name="s")
    f = pl.kernel(
        _sc_pass1_body,
        out_type=[
            jax.ShapeDtypeStruct((E // 2, 2 * HH), jnp.float32),
            jax.ShapeDtypeStruct((NW, 8, 128), jnp.float32),
            jax.ShapeDtypeStruct((NW * NP2,), jnp.float32),
        ],
        mesh=mesh,
        compiler_params=_SC_PARAMS,
        scratch_types=[
            pltpu.VMEM((CPW * CH + CH,), jnp.int32),
            pltpu.VMEM((CPW * CH + CH,), jnp.int32),
            pltpu.VMEM((NB1, CH, HH), jnp.float32),
            pltpu.VMEM((NB1, CH, HH), jnp.float32),
            pltpu.VMEM((NB1, CH // 2, 2 * HH), jnp.float32),
            pltpu.VMEM((NP2,), jnp.float32),
            pltpu.VMEM((128,), jnp.float32),
            [pltpu.SemaphoreType.DMA] * NB1,
            [pltpu.SemaphoreType.DMA] * NB1,
            [pltpu.SemaphoreType.DMA] * NB1,
        ],
    )
    return f(a, bq, dst, src)


def _tc_pass2_body(u_ref, stats_ref, e1g_ref, e1be_ref, e2w_ref, e2w2_ref,
                   e2b_ref, e2g_ref, e2be_ref, v_ref, st_ref, g_scr, sh_scr):
    i = pl.program_id(0)

    @pl.when(i == 0)
    def _():
        g_scr[...] = jnp.zeros((2 * HH, 2 * HH), jnp.float32)
        sh_scr[...] = jnp.zeros((1, 2 * HH), jnp.float32)

    parts = stats_ref[:, 0, :]
    su = jnp.sum(parts[:, :HH], axis=0, keepdims=True)
    sq = jnp.sum(parts[:, HH:], axis=0, keepdims=True)
    mu = su * (1.0 / E)
    var = sq * (1.0 / E) - mu * mu
    s1 = e1g_ref[...] * lax.rsqrt(var + EPS)
    t1 = e1be_ref[...] - mu * s1
    s1w = jnp.concatenate([s1, s1], axis=1)
    t1w = jnp.concatenate([t1, t1], axis=1)

    h1 = jnp.maximum(u_ref[...] * s1w + t1w, 0.0)
    sh_scr[...] += jnp.sum(h1, axis=0, keepdims=True)
    g_scr[...] += lax.dot_general(h1, h1, (((0,), (0,)), ((), ())),
                                  preferred_element_type=jnp.float32)
    w2 = e2w_ref[...]
    e2b = e2b_ref[...]
    e2bw = jnp.concatenate([e2b, e2b], axis=1)
    v_ref[...] = (jnp.dot(h1, e2w2_ref[...], preferred_element_type=jnp.float32)
                  + e2bw)

    shv = sh_scr[...]
    sfold = shv[:, :HH] + shv[:, HH:]
    esh = sfold * (1.0 / E)
    meanv = jnp.dot(esh, w2, preferred_element_type=jnp.float32) + e2b
    gf = g_scr[:HH, :HH] + g_scr[HH:, HH:]
    gw = jnp.dot(gf, w2, preferred_element_type=jnp.float32)
    diag = jnp.sum(w2 * gw, axis=0, keepdims=True) * (1.0 / E)
    ev2 = diag + 2.0 * e2b * (meanv - e2b) + e2b * e2b
    var2 = ev2 - meanv * meanv
    s2 = e2g_ref[...] * lax.rsqrt(var2 + EPS)
    t2 = e2be_ref[...] - meanv * s2
    st_ref[...] = jnp.concatenate([s2, t2], axis=0)


def _tc_pass2(u, stats, e1g, e1be, e2w, e2w2, e2b, e2g, e2be):
    T2R = 8000
    grid = (E // 2 // T2R,)
    small = lambda shp: pl.BlockSpec(shp, lambda i: (0, 0))
    return pl.pallas_call(
        _tc_pass2_body,
        grid=grid,
        in_specs=[
            pl.BlockSpec((T2R, 2 * HH), lambda i: (i, 0)),
            pl.BlockSpec((NW, 8, 128), lambda i: (0, 0, 0)),
            small((1, HH)), small((1, HH)),
            small((HH, HH)), small((2 * HH, 2 * HH)),
            small((1, HH)), small((1, HH)), small((1, HH)),
        ],
        out_specs=[
            pl.BlockSpec((T2R, 2 * HH), lambda i: (i, 0)),
            pl.BlockSpec((2, HH), lambda i: (0, 0)),
        ],
        out_shape=[
            jax.ShapeDtypeStruct((E // 2, 2 * HH), jnp.float32),
            jax.ShapeDtypeStruct((2, HH), jnp.float32),
        ],
        scratch_shapes=[
            pltpu.VMEM((2 * HH, 2 * HH), jnp.float32),
            pltpu.VMEM((1, 2 * HH), jnp.float32),
        ],
    )(u, stats, e1g, e1be, e2w, e2w2, e2b, e2g, e2be)


def _sc_pass3_body(v_hbm, dst_hbm, st_hbm, za_hbm,
                   agg_hbm,
                   dst2d, vrs, mbs, stv, agg_sp,
                   isem, svs, sss):
    c = lax.axis_index("c")
    s = lax.axis_index("s")
    wid = c * NS + s
    base = wid * (CPW * CH)
    rpt = NP2 // NS
    pltpu.sync_copy(za_hbm.at[pl.ds(s * rpt, rpt)], agg_sp.at[pl.ds(s * rpt, rpt)])
    pltpu.sync_copy(st_hbm, stv)

    descs = []
    for j in range(CPW):
        descs.append(pltpu.async_copy(
            dst_hbm.at[pl.ds(base + j * CH, CH)], dst2d.at[j], isem))
    for d in descs:
        d.wait()

    @pl.when(wid < TAIL_W)
    def _():
        pltpu.sync_copy(dst_hbm.at[pl.ds(TAIL_BASE + wid * CH, CH)],
                        dst2d.at[CPW])

    plsc.subcore_barrier()

    def fire_v(g, b):
        pltpu.async_copy(v_hbm.at[pl.ds((base + g * CH) // 2, CH // 2)],
                         vrs.at[b], svs[b])

    def wait_v(b):
        pltpu.make_async_copy(v_hbm.at[pl.ds(base // 2, CH // 2)], vrs.at[b],
                              svs[b]).wait()

    def wait_s(b):
        pltpu.make_async_copy(mbs.at[b], agg_sp.at[dst2d.at[0]], sss[b]).wait()

    def compute(g, b):
        vr, mb = vrs.at[b], mbs.at[b]

        def row_body(t, _):
            for half in range(2):
                for k in range(4):
                    sl = pl.ds(k * 16, 16)
                    sv = pl.ds(64 * half + k * 16, 16)
                    mb[2 * t + half, sl] = jnp.maximum(
                        vr[t, sv] * stv[0, sl] + stv[1, sl], 0.0)
            return 0

        lax.fori_loop(0, CH // 2, row_body, 0)
        pltpu.async_copy(mb, agg_sp.at[dst2d.at[g]], sss[b], add=True)

    for b in range(NB3 - 1):
        fire_v(b, b)

    def ring_body(j, _):
        for b in range(NB3):
            g = j * NB3 + b
            nxt = g + NB3 - 1

            @pl.when(nxt < CPW)
            def _():
                fire_v(nxt, (b + NB3 - 1) % NB3)

            wait_v(b)

            @pl.when(j > 0)
            def _():
                wait_s(b)

            compute(g, b)
        return 0

    lax.fori_loop(0, CPW // NB3, ring_body, 0)
    for b in range(NB3):
        wait_s(b)

    @pl.when(wid < TAIL_W)
    def _():
        tb = TAIL_BASE + wid * CH
        vr, mb = vrs.at[0], mbs.at[0]
        pltpu.sync_copy(v_hbm.at[pl.ds(tb // 2, CH // 2)], vr)

        def row_body(t, _):
            for half in range(2):
                for k in range(4):
                    sl = pl.ds(k * 16, 16)
                    sv = pl.ds(64 * half + k * 16, 16)
                    mb[2 * t + half, sl] = jnp.maximum(
                        vr[t, sv] * stv[0, sl] + stv[1, sl], 0.0)
            return 0

        lax.fori_loop(0, CH // 2, row_body, 0)
        pltpu.sync_copy(mb, agg_sp.at[dst2d.at[CPW]], add=True)

    plsc.subcore_barrier()

    @pl.when(s == 0)
    def _():
        pltpu.sync_copy(agg_sp, agg_hbm.at[c])


def _sc_pass3(v, dst, st):
    za = jnp.zeros((NP2, HH), jnp.float32)
    mesh = plsc.VectorSubcoreMesh(core_axis_name="c", subcore_axis_name="s")
    f = pl.kernel(
        _sc_pass3_body,
        out_type=jax.ShapeDtypeStruct((NC, NP2, HH), jnp.float32),
        mesh=mesh,
        compiler_params=_SC_PARAMS,
        scratch_types=[
            pltpu.VMEM((CPW + 1, CH), jnp.int32),
            pltpu.VMEM((NB3, CH // 2, 2 * HH), jnp.float32),
            pltpu.VMEM((NB3, CH, HH), jnp.float32),
            pltpu.VMEM((2, HH), jnp.float32),
            pltpu.VMEM_SHARED((NP2, HH), jnp.float32),
            pltpu.SemaphoreType.DMA,
            [pltpu.SemaphoreType.DMA] * NB3,
            [pltpu.SemaphoreType.DMA] * NB3,
        ],
    )
    return f(v, dst, st, za)


def _epi_body(agg_ref, cnt_ref, z0_ref, l1w_ref, l1b_ref, out_ref):
    agg = agg_ref[0, :N] + agg_ref[1, :N]
    cnt = jnp.zeros((1, NP2), jnp.float32)
    for w in range(NW):
        cnt = cnt + cnt_ref[:, pl.ds(w * NP2, NP2)]
    cntc = cnt[:, :N].reshape(N, 1)
    h2 = agg / jnp.maximum(cntc, 1.0)
    out_ref[...] = (z0_ref[...]
                    + jnp.dot(h2, l1w_ref[...], preferred_element_type=jnp.float32)
                    + l1b_ref[...])


def _epilogue(agg2, cnt_flat, z0, l1w, l1b):
    return pl.pallas_call(
        _epi_body,
        out_shape=jax.ShapeDtypeStruct((N, 2), jnp.float32),
    )(agg2, cnt_flat.reshape(1, NW * NP2), z0, l1w, l1b)


def kernel(x, paramsE, paramsS, edge_index):
    pE, pS = paramsE, paramsS
    cat = lambda k: jnp.concatenate([pE[k], pS[k]])[None]
    f1w = jnp.concatenate([pE['f1_w'], pS['f1_w']], axis=1)
    f2w = _blkdiag(pE['f2_w'], pS['f2_w'])
    l0w = _blkdiag(pE['l0_w'], pS['l0_w'])
    l1w = _blkdiag(pE['l1_w'], pS['l1_w'])
    pw = _blkdiag(pE['e1_w'][:H] - pE['e1_w'][H:], pS['e1_w'][:H] - pS['e1_w'][H:])
    qw = _blkdiag(pE['e1_w'][H:], pS['e1_w'][H:])
    e2w = _blkdiag(pE['e2_w'], pS['e2_w'])

    a, bq, z0 = _node_prep(
        x, f1w, cat('f1_b'), cat('f1_g'), cat('f1_be'),
        f2w, cat('f2_b'), cat('f2_g'), cat('f2_be'),
        l0w, cat('l0_b'), pw, qw)

    src = edge_index[0].astype(jnp.int32)
    dst = edge_index[1].astype(jnp.int32)

    u, stats, cnt_flat = _sc_pass1(a, bq, dst, src)
    e2w2 = _blkdiag(e2w, e2w)
    v, st = _tc_pass2(u, stats, cat('e1_g'), cat('e1_be'),
                      e2w, e2w2, cat('e2_b'), cat('e2_g'), cat('e2_be'))
    agg2 = _sc_pass3(v, dst, st)
    return _epilogue(agg2, cnt_flat, z0, l1w, cat('l1_b'))


# speedup vs baseline: 1.3211x; 1.0090x over previous
import functools

import jax
import jax.numpy as jnp
from jax import lax
from jax.experimental import pallas as pl
from jax.experimental.pallas import tpu as pltpu
from jax.experimental.pallas import tpu_sc as plsc

N = 10000
E = 320000
D = 128
H = 32
HH = 2 * H
NC, NS = 2, 16
NW = NC * NS
CH = 128
CPW = 78
TAIL_W = (E - NW * CPW * CH) // CH
TAIL_BASE = NW * CPW * CH
NP2 = 10240
EPS = 1e-5
T2 = 8000
NB1 = 3
NB3 = 3

_SC_PARAMS = pltpu.CompilerParams(use_tc_tiling_on_sc=False,
                                  needs_layout_passes=False)


def _blkdiag(A, B):
    za = jnp.zeros((A.shape[0], B.shape[1]), A.dtype)
    zb = jnp.zeros((B.shape[0], A.shape[1]), B.dtype)
    return jnp.concatenate(
        [jnp.concatenate([A, za], axis=1), jnp.concatenate([zb, B], axis=1)], axis=0)


def _node_prep_body(x_ref, f1w_ref, f1b_ref, f1g_ref, f1be_ref,
                    f2w_ref, f2b_ref, f2g_ref, f2be_ref,
                    l0w_ref, l0b_ref, pw_ref, qw_ref,
                    a_ref, bq_ref, z0_ref):
    def bn_relu(h, g, be):
        m = jnp.mean(h, axis=0, keepdims=True)
        v = jnp.mean((h - m) * (h - m), axis=0, keepdims=True)
        return jnp.maximum((h - m) * lax.rsqrt(v + EPS) * g + be, 0.0)

    h = jnp.dot(x_ref[...], f1w_ref[...], preferred_element_type=jnp.float32)
    h = bn_relu(h + f1b_ref[...], f1g_ref[...], f1be_ref[...])
    h = jnp.dot(h, f2w_ref[...], preferred_element_type=jnp.float32)
    h = bn_relu(h + f2b_ref[...], f2g_ref[...], f2be_ref[...])
    z0_ref[...] = (jnp.dot(h, l0w_ref[...], preferred_element_type=jnp.float32)
                   + l0b_ref[...])
    a_ref[...] = jnp.dot(h, pw_ref[...], preferred_element_type=jnp.float32)
    bq_ref[...] = jnp.dot(h, qw_ref[...], preferred_element_type=jnp.float32)


def _node_prep(x, f1w, f1b, f1g, f1be, f2w, f2b, f2g, f2be, l0w, l0b, pw, qw):
    return pl.pallas_call(
        _node_prep_body,
        out_shape=[
            jax.ShapeDtypeStruct((N, HH), jnp.float32),
            jax.ShapeDtypeStruct((N, HH), jnp.float32),
            jax.ShapeDtypeStruct((N, 2), jnp.float32),
        ],
    )(x, f1w, f1b, f1g, f1be, f2w, f2b, f2g, f2be, l0w, l0b, pw, qw)


def _sc_pass1_body(a_hbm, bq_hbm, dst_hbm, src_hbm,
                   u_hbm, stats_hbm, cnt_hbm,
                   dstall, srcall, ars, brs, ubs, hist, accv,
                   sgas, sgbs, sos):
    c = lax.axis_index("c")
    s = lax.axis_index("s")
    wid = c * NS + s
    base = wid * (CPW * CH)
    npre = CPW * CH
    ones16 = jnp.ones((16,), jnp.float32)
    zero16 = jnp.zeros((16,), jnp.float32)

    pltpu.sync_copy(dst_hbm.at[pl.ds(base, npre)], dstall.at[pl.ds(0, npre)])
    pltpu.sync_copy(src_hbm.at[pl.ds(base, npre)], srcall.at[pl.ds(0, npre)])

    @pl.when(wid < TAIL_W)
    def _():
        tb = TAIL_BASE + wid * CH
        pltpu.sync_copy(dst_hbm.at[pl.ds(tb, CH)], dstall.at[pl.ds(npre, CH)])
        pltpu.sync_copy(src_hbm.at[pl.ds(tb, CH)], srcall.at[pl.ds(npre, CH)])

    def zb(j, _):
        hist[pl.ds(j * 16, 16)] = zero16
        return 0
    lax.fori_loop(0, NP2 // 16, zb, 0)

    def fire(g, b):
        pltpu.async_copy(a_hbm.at[dstall.at[pl.ds(g * CH, CH)]], ars.at[b], sgas[b])
        pltpu.async_copy(bq_hbm.at[srcall.at[pl.ds(g * CH, CH)]], brs.at[b], sgbs[b])

    def wait_gather(b):
        pltpu.make_async_copy(a_hbm.at[dstall.at[pl.ds(0, CH)]], ars.at[b], sgas[b]).wait()
        pltpu.make_async_copy(bq_hbm.at[srcall.at[pl.ds(0, CH)]], brs.at[b], sgbs[b]).wait()

    def wait_out(b):
        pltpu.make_async_copy(ubs.at[b], u_hbm.at[pl.ds(base // 2, CH // 2)],
                              sos[b]).wait()

    def compute(g, b, accs):
        ar, br, ub = ars.at[b], brs.at[b], ubs.at[b]

        def row_body(t, acc):
            s0, s1, s2, s3, q0, q1, q2, q3 = acc
            r0 = 2 * t
            r1 = 2 * t + 1
            a0 = ar[r0, pl.ds(0, 16)] + br[r0, pl.ds(0, 16)]
            a1 = ar[r0, pl.ds(16, 16)] + br[r0, pl.ds(16, 16)]
            a2 = ar[r0, pl.ds(32, 16)] + br[r0, pl.ds(32, 16)]
            a3 = ar[r0, pl.ds(48, 16)] + br[r0, pl.ds(48, 16)]
            b0 = ar[r1, pl.ds(0, 16)] + br[r1, pl.ds(0, 16)]
            b1 = ar[r1, pl.ds(16, 16)] + br[r1, pl.ds(16, 16)]
            b2 = ar[r1, pl.ds(32, 16)] + br[r1, pl.ds(32, 16)]
            b3 = ar[r1, pl.ds(48, 16)] + br[r1, pl.ds(48, 16)]
            ub[t, pl.ds(0, 16)] = a0
            ub[t, pl.ds(16, 16)] = a1
            ub[t, pl.ds(32, 16)] = a2
            ub[t, pl.ds(48, 16)] = a3
            ub[t, pl.ds(64, 16)] = b0
            ub[t, pl.ds(80, 16)] = b1
            ub[t, pl.ds(96, 16)] = b2
            ub[t, pl.ds(112, 16)] = b3
            return (s0 + a0 + b0, s1 + a1 + b1, s2 + a2 + b2, s3 + a3 + b3,
                    q0 + a0 * a0 + b0 * b0, q1 + a1 * a1 + b1 * b1,
                    q2 + a2 * a2 + b2 * b2, q3 + a3 * a3 + b3 * b3)

        accs = lax.fori_loop(0, CH // 2, row_body, accs)
        pltpu.async_copy(ub, u_hbm.at[pl.ds((base + g * CH) // 2, CH // 2)],
                         sos[b])
        for j in range(CH // 16):
            iv = dstall[pl.ds(g * CH + j * 16, 16)]
            plsc.addupdate_scatter(hist, [iv], ones16)
        return accs

    accs0 = (zero16,) * 8
    for b in range(NB1 - 1):
        fire(b, b)

    def ring_body(j, accs):
        for b in range(NB1):
            g = j * NB1 + b
            nxt = g + NB1 - 1

            @pl.when(nxt < CPW)
            def _():
                fire(nxt, (b + NB1 - 1) % NB1)

            wait_gather(b)

            @pl.when(j > 0)
            def _():
                wait_out(b)

            accs = compute(g, b, accs)
        return accs

    accs = lax.fori_loop(0, CPW // NB1, ring_body, accs0)

    for b in range(NB1):
        wait_out(b)
    for k in range(8):
        accv[pl.ds(k * 16, 16)] = accs[k]

    @pl.when(wid < TAIL_W)
    def _():
        tb = TAIL_BASE + wid * CH
        ar, br, ub = ars.at[0], brs.at[0], ubs.at[0]
        cp1 = pltpu.async_copy(a_hbm.at[dstall.at[pl.ds(npre, CH)]], ar, sgas[0])
        cp2 = pltpu.async_copy(bq_hbm.at[srcall.at[pl.ds(npre, CH)]], br, sgbs[0])
        cp1.wait()
        cp2.wait()

        def row_body(t, _):
            for half in range(2):
                r = 2 * t + half
                for k in range(4):
                    sl = pl.ds(k * 16, 16)
                    uv = ar[r, sl] + br[r, sl]
                    ub[t, pl.ds(64 * half + k * 16, 16)] = uv
                    sa = pl.ds(k * 16, 16)
                    accv[sa] = accv[sa] + uv
                    sb = pl.ds(64 + k * 16, 16)
                    accv[sb] = accv[sb] + uv * uv
            return 0

        lax.fori_loop(0, CH // 2, row_body, 0)
        pltpu.sync_copy(ub, u_hbm.at[pl.ds(tb // 2, CH // 2)])
        for j in range(CH // 16):
            iv = dstall[pl.ds(npre + j * 16, 16)]
            plsc.addupdate_scatter(hist, [iv], ones16)

    pltpu.sync_copy(accv, stats_hbm.at[wid, 0])
    pltpu.sync_copy(hist, cnt_hbm.at[pl.ds(wid * NP2, NP2)])


def _sc_pass1(a, bq, dst, src):
    mesh = plsc.VectorSubcoreMesh(core_axis_name="c", subcore_axis_name="s")
    f = pl.kernel(
        _sc_pass1_body,
        out_type=[
            jax.ShapeDtypeStruct((E // 2, 2 * HH), jnp.float32),
            jax.ShapeDtypeStruct((NW, 8, 128), jnp.float32),
            jax.ShapeDtypeStruct((NW * NP2,), jnp.float32),
        ],
        mesh=mesh,
        compiler_params=_SC_PARAMS,
        scratch_types=[
            pltpu.VMEM((CPW * CH + CH,), jnp.int32),
            pltpu.VMEM((CPW * CH + CH,), jnp.int32),
            pltpu.VMEM((NB1, CH, HH), jnp.float32),
            pltpu.VMEM((NB1, CH, HH), jnp.float32),
            pltpu.VMEM((NB1, CH // 2, 2 * HH), jnp.float32),
            pltpu.VMEM((NP2,), jnp.float32),
            pltpu.VMEM((128,), jnp.float32),
            [pltpu.SemaphoreType.DMA] * NB1,
            [pltpu.SemaphoreType.DMA] * NB1,
            [pltpu.SemaphoreType.DMA] * NB1,
        ],
    )
    return f(a, bq, dst, src)


def _tc_pass2_body(u_ref, stats_ref, e1g_ref, e1be_ref, e2w_ref, e2w2_ref,
                   e2b_ref, e2g_ref, e2be_ref, v_ref, st_ref, g_scr, sh_scr):
    i = pl.program_id(0)

    @pl.when(i == 0)
    def _():
        g_scr[...] = jnp.zeros((2 * HH, 2 * HH), jnp.float32)
        sh_scr[...] = jnp.zeros((1, 2 * HH), jnp.float32)

    parts = stats_ref[:, 0, :]
    su = jnp.sum(parts[:, :HH], axis=0, keepdims=True)
    sq = jnp.sum(parts[:, HH:], axis=0, keepdims=True)
    mu = su * (1.0 / E)
    var = sq * (1.0 / E) - mu * mu
    s1 = e1g_ref[...] * lax.rsqrt(var + EPS)
    t1 = e1be_ref[...] - mu * s1
    s1w = jnp.concatenate([s1, s1], axis=1)
    t1w = jnp.concatenate([t1, t1], axis=1)

    h1 = jnp.maximum(u_ref[...] * s1w + t1w, 0.0)
    sh_scr[...] += jnp.sum(h1, axis=0, keepdims=True)
    g_scr[...] += lax.dot_general(h1, h1, (((0,), (0,)), ((), ())),
                                  preferred_element_type=jnp.float32)
    w2 = e2w_ref[...]
    e2b = e2b_ref[...]
    e2bw = jnp.concatenate([e2b, e2b], axis=1)
    v_ref[...] = (jnp.dot(h1, e2w2_ref[...], preferred_element_type=jnp.float32)
                  + e2bw)

    shv = sh_scr[...]
    sfold = shv[:, :HH] + shv[:, HH:]
    esh = sfold * (1.0 / E)
    meanv = jnp.dot(esh, w2, preferred_element_type=jnp.float32) + e2b
    gf = g_scr[:HH, :HH] + g_scr[HH:, HH:]
    gw = jnp.dot(gf, w2, preferred_element_type=jnp.float32)
    diag = jnp.sum(w2 * gw, axis=0, keepdims=True) * (1.0 / E)
    ev2 = diag + 2.0 * e2b * (meanv - e2b) + e2b * e2b
    var2 = ev2 - meanv * meanv
    s2 = e2g_ref[...] * lax.rsqrt(var2 + EPS)
    t2 = e2be_ref[...] - meanv * s2
    st_ref[...] = jnp.concatenate([s2, t2], axis=0)


def _tc_pass2(u, stats, e1g, e1be, e2w, e2w2, e2b, e2g, e2be):
    T2R = 16000
    grid = (E // 2 // T2R,)
    small = lambda shp: pl.BlockSpec(shp, lambda i: (0, 0))
    return pl.pallas_call(
        _tc_pass2_body,
        grid=grid,
        in_specs=[
            pl.BlockSpec((T2R, 2 * HH), lambda i: (i, 0)),
            pl.BlockSpec((NW, 8, 128), lambda i: (0, 0, 0)),
            small((1, HH)), small((1, HH)),
            small((HH, HH)), small((2 * HH, 2 * HH)),
            small((1, HH)), small((1, HH)), small((1, HH)),
        ],
        out_specs=[
            pl.BlockSpec((T2R, 2 * HH), lambda i: (i, 0)),
            pl.BlockSpec((2, HH), lambda i: (0, 0)),
        ],
        out_shape=[
            jax.ShapeDtypeStruct((E // 2, 2 * HH), jnp.float32),
            jax.ShapeDtypeStruct((2, HH), jnp.float32),
        ],
        scratch_shapes=[
            pltpu.VMEM((2 * HH, 2 * HH), jnp.float32),
            pltpu.VMEM((1, 2 * HH), jnp.float32),
        ],
    )(u, stats, e1g, e1be, e2w, e2w2, e2b, e2g, e2be)


def _sc_pass3_body(v_hbm, dst_hbm, st_hbm, za_hbm,
                   agg_hbm,
                   dst2d, vrs, mbs, stv, agg_sp,
                   isem, svs, sss):
    c = lax.axis_index("c")
    s = lax.axis_index("s")
    wid = c * NS + s
    base = wid * (CPW * CH)
    rpt = NP2 // NS
    pltpu.sync_copy(za_hbm.at[pl.ds(s * rpt, rpt)], agg_sp.at[pl.ds(s * rpt, rpt)])
    pltpu.sync_copy(st_hbm, stv)

    descs = []
    for j in range(CPW):
        descs.append(pltpu.async_copy(
            dst_hbm.at[pl.ds(base + j * CH, CH)], dst2d.at[j], isem))
    for d in descs:
        d.wait()

    @pl.when(wid < TAIL_W)
    def _():
        pltpu.sync_copy(dst_hbm.at[pl.ds(TAIL_BASE + wid * CH, CH)],
                        dst2d.at[CPW])

    plsc.subcore_barrier()

    def fire_v(g, b):
        pltpu.async_copy(v_hbm.at[pl.ds((base + g * CH) // 2, CH // 2)],
                         vrs.at[b], svs[b])

    def wait_v(b):
        pltpu.make_async_copy(v_hbm.at[pl.ds(base // 2, CH // 2)], vrs.at[b],
                              svs[b]).wait()

    def wait_s(b):
        pltpu.make_async_copy(mbs.at[b], agg_sp.at[dst2d.at[0]], sss[b]).wait()

    def compute(g, b):
        vr, mb = vrs.at[b], mbs.at[b]

        def row_body(t, _):
            for half in range(2):
                for k in range(4):
                    sl = pl.ds(k * 16, 16)
                    sv = pl.ds(64 * half + k * 16, 16)
                    mb[2 * t + half, sl] = jnp.maximum(
                        vr[t, sv] * stv[0, sl] + stv[1, sl], 0.0)
            return 0

        lax.fori_loop(0, CH // 2, row_body, 0)
        pltpu.async_copy(mb, agg_sp.at[dst2d.at[g]], sss[b], add=True)

    for b in range(NB3 - 1):
        fire_v(b, b)

    def ring_body(j, _):
        for b in range(NB3):
            g = j * NB3 + b
            nxt = g + NB3 - 1

            @pl.when(nxt < CPW)
            def _():
                fire_v(nxt, (b + NB3 - 1) % NB3)

            wait_v(b)

            @pl.when(j > 0)
            def _():
                wait_s(b)

            compute(g, b)
        return 0

    lax.fori_loop(0, CPW // NB3, ring_body, 0)
    for b in range(NB3):
        wait_s(b)

    @pl.when(wid < TAIL_W)
    def _():
        tb = TAIL_BASE + wid * CH
        vr, mb = vrs.at[0], mbs.at[0]
        pltpu.sync_copy(v_hbm.at[pl.ds(tb // 2, CH // 2)], vr)

        def row_body(t, _):
            for half in range(2):
                for k in range(4):
                    sl = pl.ds(k * 16, 16)
                    sv = pl.ds(64 * half + k * 16, 16)
                    mb[2 * t + half, sl] = jnp.maximum(
                        vr[t, sv] * stv[0, sl] + stv[1, sl], 0.0)
            return 0

        lax.fori_loop(0, CH // 2, row_body, 0)
        pltpu.sync_copy(mb, agg_sp.at[dst2d.at[CPW]], add=True)

    plsc.subcore_barrier()

    @pl.when(s == 0)
    def _():
        pltpu.sync_copy(agg_sp, agg_hbm.at[c])


def _sc_pass3(v, dst, st):
    za = jnp.zeros((NP2, HH), jnp.float32)
    mesh = plsc.VectorSubcoreMesh(core_axis_name="c", subcore_axis_name="s")
    f = pl.kernel(
        _sc_pass3_body,
        out_type=jax.ShapeDtypeStruct((NC, NP2, HH), jnp.float32),
        mesh=mesh,
        compiler_params=_SC_PARAMS,
        scratch_types=[
            pltpu.VMEM((CPW + 1, CH), jnp.int32),
            pltpu.VMEM((NB3, CH // 2, 2 * HH), jnp.float32),
            pltpu.VMEM((NB3, CH, HH), jnp.float32),
            pltpu.VMEM((2, HH), jnp.float32),
            pltpu.VMEM_SHARED((NP2, HH), jnp.float32),
            pltpu.SemaphoreType.DMA,
            [pltpu.SemaphoreType.DMA] * NB3,
            [pltpu.SemaphoreType.DMA] * NB3,
        ],
    )
    return f(v, dst, st, za)


def _epi_body(agg_ref, cnt_ref, z0_ref, l1w_ref, l1b_ref, out_ref):
    agg = agg_ref[0, :N] + agg_ref[1, :N]
    cnt = jnp.zeros((1, NP2), jnp.float32)
    for w in range(NW):
        cnt = cnt + cnt_ref[:, pl.ds(w * NP2, NP2)]
    cntc = cnt[:, :N].reshape(N, 1)
    h2 = agg / jnp.maximum(cntc, 1.0)
    out_ref[...] = (z0_ref[...]
                    + jnp.dot(h2, l1w_ref[...], preferred_element_type=jnp.float32)
                    + l1b_ref[...])


def _epilogue(agg2, cnt_flat, z0, l1w, l1b):
    return pl.pallas_call(
        _epi_body,
        out_shape=jax.ShapeDtypeStruct((N, 2), jnp.float32),
    )(agg2, cnt_flat.reshape(1, NW * NP2), z0, l1w, l1b)


def kernel(x, paramsE, paramsS, edge_index):
    pE, pS = paramsE, paramsS
    cat = lambda k: jnp.concatenate([pE[k], pS[k]])[None]
    f1w = jnp.concatenate([pE['f1_w'], pS['f1_w']], axis=1)
    f2w = _blkdiag(pE['f2_w'], pS['f2_w'])
    l0w = _blkdiag(pE['l0_w'], pS['l0_w'])
    l1w = _blkdiag(pE['l1_w'], pS['l1_w'])
    pw = _blkdiag(pE['e1_w'][:H] - pE['e1_w'][H:], pS['e1_w'][:H] - pS['e1_w'][H:])
    qw = _blkdiag(pE['e1_w'][H:], pS['e1_w'][H:])
    e2w = _blkdiag(pE['e2_w'], pS['e2_w'])

    a, bq, z0 = _node_prep(
        x, f1w, cat('f1_b'), cat('f1_g'), cat('f1_be'),
        f2w, cat('f2_b'), cat('f2_g'), cat('f2_be'),
        l0w, cat('l0_b'), pw, qw)

    src = edge_index[0].astype(jnp.int32)
    dst = edge_index[1].astype(jnp.int32)

    u, stats, cnt_flat = _sc_pass1(a, bq, dst, src)
    e2w2 = _blkdiag(e2w, e2w)
    v, st = _tc_pass2(u, stats, cat('e1_g'), cat('e1_be'),
                      e2w, e2w2, cat('e2_b'), cat('e2_g'), cat('e2_be'))
    agg2 = _sc_pass3(v, dst, st)
    return _epilogue(agg2, cnt_flat, z0, l1w, cat('l1_b'))


# speedup vs baseline: 1.3230x; 1.0014x over previous
import functools

import jax
import jax.numpy as jnp
from jax import lax
from jax.experimental import pallas as pl
from jax.experimental.pallas import tpu as pltpu
from jax.experimental.pallas import tpu_sc as plsc

N = 10000
E = 320000
D = 128
H = 32
HH = 2 * H
NC, NS = 2, 16
NW = NC * NS
CH = 128
CPW = 78
TAIL_W = (E - NW * CPW * CH) // CH
TAIL_BASE = NW * CPW * CH
NP2 = 10240
EPS = 1e-5
T2 = 8000
NB1 = 3
NB3 = 3

_SC_PARAMS = pltpu.CompilerParams(use_tc_tiling_on_sc=False,
                                  needs_layout_passes=False)


def _blkdiag(A, B):
    za = jnp.zeros((A.shape[0], B.shape[1]), A.dtype)
    zb = jnp.zeros((B.shape[0], A.shape[1]), B.dtype)
    return jnp.concatenate(
        [jnp.concatenate([A, za], axis=1), jnp.concatenate([zb, B], axis=1)], axis=0)


def _node_prep_body(x_ref, f1w_ref, f1b_ref, f1g_ref, f1be_ref,
                    f2w_ref, f2b_ref, f2g_ref, f2be_ref,
                    l0w_ref, l0b_ref, pw_ref, qw_ref,
                    a_ref, bq_ref, z0_ref):
    def bn_relu(h, g, be):
        m = jnp.mean(h, axis=0, keepdims=True)
        v = jnp.mean((h - m) * (h - m), axis=0, keepdims=True)
        return jnp.maximum((h - m) * lax.rsqrt(v + EPS) * g + be, 0.0)

    h = jnp.dot(x_ref[...], f1w_ref[...], preferred_element_type=jnp.float32)
    h = bn_relu(h + f1b_ref[...], f1g_ref[...], f1be_ref[...])
    h = jnp.dot(h, f2w_ref[...], preferred_element_type=jnp.float32)
    h = bn_relu(h + f2b_ref[...], f2g_ref[...], f2be_ref[...])
    z0_ref[...] = (jnp.dot(h, l0w_ref[...], preferred_element_type=jnp.float32)
                   + l0b_ref[...])
    a_ref[...] = jnp.dot(h, pw_ref[...], preferred_element_type=jnp.float32)
    bq_ref[...] = jnp.dot(h, qw_ref[...], preferred_element_type=jnp.float32)


def _node_prep(x, f1w, f1b, f1g, f1be, f2w, f2b, f2g, f2be, l0w, l0b, pw, qw):
    return pl.pallas_call(
        _node_prep_body,
        out_shape=[
            jax.ShapeDtypeStruct((N, HH), jnp.float32),
            jax.ShapeDtypeStruct((N, HH), jnp.float32),
            jax.ShapeDtypeStruct((N, 2), jnp.float32),
        ],
    )(x, f1w, f1b, f1g, f1be, f2w, f2b, f2g, f2be, l0w, l0b, pw, qw)


def _sc_pass1_body(a_hbm, bq_hbm, dst_hbm, src_hbm,
                   u_hbm, stats_hbm, cnt_hbm,
                   dstall, srcall, ars, brs, ubs, hist, accv,
                   sgas, sgbs, sos):
    c = lax.axis_index("c")
    s = lax.axis_index("s")
    wid = c * NS + s
    base = wid * (CPW * CH)
    npre = CPW * CH
    ones16 = jnp.ones((16,), jnp.float32)
    zero16 = jnp.zeros((16,), jnp.float32)

    pltpu.sync_copy(dst_hbm.at[pl.ds(base, npre)], dstall.at[pl.ds(0, npre)])
    pltpu.sync_copy(src_hbm.at[pl.ds(base, npre)], srcall.at[pl.ds(0, npre)])

    @pl.when(wid < TAIL_W)
    def _():
        tb = TAIL_BASE + wid * CH
        pltpu.sync_copy(dst_hbm.at[pl.ds(tb, CH)], dstall.at[pl.ds(npre, CH)])
        pltpu.sync_copy(src_hbm.at[pl.ds(tb, CH)], srcall.at[pl.ds(npre, CH)])

    def zb(j, _):
        hist[pl.ds(j * 16, 16)] = zero16
        return 0
    lax.fori_loop(0, NP2 // 16, zb, 0)

    def fire(g, b):
        pltpu.async_copy(a_hbm.at[dstall.at[pl.ds(g * CH, CH)]], ars.at[b], sgas[b])
        pltpu.async_copy(bq_hbm.at[srcall.at[pl.ds(g * CH, CH)]], brs.at[b], sgbs[b])

    def wait_gather(b):
        pltpu.make_async_copy(a_hbm.at[dstall.at[pl.ds(0, CH)]], ars.at[b], sgas[b]).wait()
        pltpu.make_async_copy(bq_hbm.at[srcall.at[pl.ds(0, CH)]], brs.at[b], sgbs[b]).wait()

    def wait_out(b):
        pltpu.make_async_copy(ubs.at[b], u_hbm.at[pl.ds(base // 2, CH // 2)],
                              sos[b]).wait()

    def compute(g, b, accs):
        ar, br, ub = ars.at[b], brs.at[b], ubs.at[b]

        def row_body(t, acc):
            s0, s1, s2, s3, q0, q1, q2, q3 = acc
            r0 = 2 * t
            r1 = 2 * t + 1
            a0 = ar[r0, pl.ds(0, 16)] + br[r0, pl.ds(0, 16)]
            a1 = ar[r0, pl.ds(16, 16)] + br[r0, pl.ds(16, 16)]
            a2 = ar[r0, pl.ds(32, 16)] + br[r0, pl.ds(32, 16)]
            a3 = ar[r0, pl.ds(48, 16)] + br[r0, pl.ds(48, 16)]
            b0 = ar[r1, pl.ds(0, 16)] + br[r1, pl.ds(0, 16)]
            b1 = ar[r1, pl.ds(16, 16)] + br[r1, pl.ds(16, 16)]
            b2 = ar[r1, pl.ds(32, 16)] + br[r1, pl.ds(32, 16)]
            b3 = ar[r1, pl.ds(48, 16)] + br[r1, pl.ds(48, 16)]
            ub[t, pl.ds(0, 16)] = a0
            ub[t, pl.ds(16, 16)] = a1
            ub[t, pl.ds(32, 16)] = a2
            ub[t, pl.ds(48, 16)] = a3
            ub[t, pl.ds(64, 16)] = b0
            ub[t, pl.ds(80, 16)] = b1
            ub[t, pl.ds(96, 16)] = b2
            ub[t, pl.ds(112, 16)] = b3
            return (s0 + a0 + b0, s1 + a1 + b1, s2 + a2 + b2, s3 + a3 + b3,
                    q0 + a0 * a0 + b0 * b0, q1 + a1 * a1 + b1 * b1,
                    q2 + a2 * a2 + b2 * b2, q3 + a3 * a3 + b3 * b3)

        accs = lax.fori_loop(0, CH // 2, row_body, accs)
        pltpu.async_copy(ub, u_hbm.at[pl.ds((base + g * CH) // 2, CH // 2)],
                         sos[b])
        for j in range(CH // 16):
            iv = dstall[pl.ds(g * CH + j * 16, 16)]
            plsc.addupdate_scatter(hist, [iv], ones16)
        return accs

    accs0 = (zero16,) * 8
    for b in range(NB1 - 1):
        fire(b, b)

    def ring_body(j, accs):
        for b in range(NB1):
            g = j * NB1 + b
            nxt = g + NB1 - 1

            @pl.when(nxt < CPW)
            def _():
                fire(nxt, (b + NB1 - 1) % NB1)

            wait_gather(b)

            @pl.when(j > 0)
            def _():
                wait_out(b)

            accs = compute(g, b, accs)
        return accs

    accs = lax.fori_loop(0, CPW // NB1, ring_body, accs0)

    for b in range(NB1):
        wait_out(b)
    for k in range(8):
        accv[pl.ds(k * 16, 16)] = accs[k]

    @pl.when(wid < TAIL_W)
    def _():
        tb = TAIL_BASE + wid * CH
        ar, br, ub = ars.at[0], brs.at[0], ubs.at[0]
        cp1 = pltpu.async_copy(a_hbm.at[dstall.at[pl.ds(npre, CH)]], ar, sgas[0])
        cp2 = pltpu.async_copy(bq_hbm.at[srcall.at[pl.ds(npre, CH)]], br, sgbs[0])
        cp1.wait()
        cp2.wait()

        def row_body(t, _):
            for half in range(2):
                r = 2 * t + half
                for k in range(4):
                    sl = pl.ds(k * 16, 16)
                    uv = ar[r, sl] + br[r, sl]
                    ub[t, pl.ds(64 * half + k * 16, 16)] = uv
                    sa = pl.ds(k * 16, 16)
                    accv[sa] = accv[sa] + uv
                    sb = pl.ds(64 + k * 16, 16)
                    accv[sb] = accv[sb] + uv * uv
            return 0

        lax.fori_loop(0, CH // 2, row_body, 0)
        pltpu.sync_copy(ub, u_hbm.at[pl.ds(tb // 2, CH // 2)])
        for j in range(CH // 16):
            iv = dstall[pl.ds(npre + j * 16, 16)]
            plsc.addupdate_scatter(hist, [iv], ones16)

    pltpu.sync_copy(accv, stats_hbm.at[wid, 0])
    pltpu.sync_copy(hist, cnt_hbm.at[pl.ds(wid * NP2, NP2)])


def _sc_pass1(a, bq, dst, src):
    mesh = plsc.VectorSubcoreMesh(core_axis_name="c", subcore_axis_name="s")
    f = pl.kernel(
        _sc_pass1_body,
        out_type=[
            jax.ShapeDtypeStruct((E // 2, 2 * HH), jnp.float32),
            jax.ShapeDtypeStruct((NW, 8, 128), jnp.float32),
            jax.ShapeDtypeStruct((NW * NP2,), jnp.float32),
        ],
        mesh=mesh,
        compiler_params=_SC_PARAMS,
        scratch_types=[
            pltpu.VMEM((CPW * CH + CH,), jnp.int32),
            pltpu.VMEM((CPW * CH + CH,), jnp.int32),
            pltpu.VMEM((NB1, CH, HH), jnp.float32),
            pltpu.VMEM((NB1, CH, HH), jnp.float32),
            pltpu.VMEM((NB1, CH // 2, 2 * HH), jnp.float32),
            pltpu.VMEM((NP2,), jnp.float32),
            pltpu.VMEM((128,), jnp.float32),
            [pltpu.SemaphoreType.DMA] * NB1,
            [pltpu.SemaphoreType.DMA] * NB1,
            [pltpu.SemaphoreType.DMA] * NB1,
        ],
    )
    return f(a, bq, dst, src)


def _tc_pass2_body(u_ref, stats_ref, e1g_ref, e1be_ref, e2w_ref, e2w2_ref,
                   e2b_ref, e2g_ref, e2be_ref, v_ref, st_ref, g_scr, sh_scr):
    i = pl.program_id(0)

    @pl.when(i == 0)
    def _():
        g_scr[...] = jnp.zeros((2 * HH, 2 * HH), jnp.float32)
        sh_scr[...] = jnp.zeros((1, 2 * HH), jnp.float32)

    parts = stats_ref[:, 0, :]
    su = jnp.sum(parts[:, :HH], axis=0, keepdims=True)
    sq = jnp.sum(parts[:, HH:], axis=0, keepdims=True)
    mu = su * (1.0 / E)
    var = sq * (1.0 / E) - mu * mu
    s1 = e1g_ref[...] * lax.rsqrt(var + EPS)
    t1 = e1be_ref[...] - mu * s1
    s1w = jnp.concatenate([s1, s1], axis=1)
    t1w = jnp.concatenate([t1, t1], axis=1)

    h1 = jnp.maximum(u_ref[...] * s1w + t1w, 0.0)
    sh_scr[...] += jnp.sum(h1, axis=0, keepdims=True)
    g_scr[...] += lax.dot_general(h1, h1, (((0,), (0,)), ((), ())),
                                  preferred_element_type=jnp.float32)
    w2 = e2w_ref[...]
    e2b = e2b_ref[...]
    e2bw = jnp.concatenate([e2b, e2b], axis=1)
    v_ref[...] = (jnp.dot(h1, e2w2_ref[...], preferred_element_type=jnp.float32)
                  + e2bw)

    shv = sh_scr[...]
    sfold = shv[:, :HH] + shv[:, HH:]
    esh = sfold * (1.0 / E)
    meanv = jnp.dot(esh, w2, preferred_element_type=jnp.float32) + e2b
    gf = g_scr[:HH, :HH] + g_scr[HH:, HH:]
    gw = jnp.dot(gf, w2, preferred_element_type=jnp.float32)
    diag = jnp.sum(w2 * gw, axis=0, keepdims=True) * (1.0 / E)
    ev2 = diag + 2.0 * e2b * (meanv - e2b) + e2b * e2b
    var2 = ev2 - meanv * meanv
    s2 = e2g_ref[...] * lax.rsqrt(var2 + EPS)
    t2 = e2be_ref[...] - meanv * s2
    st_ref[...] = jnp.concatenate([s2, t2], axis=0)


def _tc_pass2(u, stats, e1g, e1be, e2w, e2w2, e2b, e2g, e2be):
    T2R = 20000
    grid = (E // 2 // T2R,)
    small = lambda shp: pl.BlockSpec(shp, lambda i: (0, 0))
    return pl.pallas_call(
        _tc_pass2_body,
        grid=grid,
        in_specs=[
            pl.BlockSpec((T2R, 2 * HH), lambda i: (i, 0)),
            pl.BlockSpec((NW, 8, 128), lambda i: (0, 0, 0)),
            small((1, HH)), small((1, HH)),
            small((HH, HH)), small((2 * HH, 2 * HH)),
            small((1, HH)), small((1, HH)), small((1, HH)),
        ],
        out_specs=[
            pl.BlockSpec((T2R, 2 * HH), lambda i: (i, 0)),
            pl.BlockSpec((2, HH), lambda i: (0, 0)),
        ],
        out_shape=[
            jax.ShapeDtypeStruct((E // 2, 2 * HH), jnp.float32),
            jax.ShapeDtypeStruct((2, HH), jnp.float32),
        ],
        scratch_shapes=[
            pltpu.VMEM((2 * HH, 2 * HH), jnp.float32),
            pltpu.VMEM((1, 2 * HH), jnp.float32),
        ],
    )(u, stats, e1g, e1be, e2w, e2w2, e2b, e2g, e2be)


def _sc_pass3_body(v_hbm, dst_hbm, st_hbm, za_hbm,
                   agg_hbm,
                   dst2d, vrs, mbs, stv, agg_sp,
                   isem, svs, sss):
    c = lax.axis_index("c")
    s = lax.axis_index("s")
    wid = c * NS + s
    base = wid * (CPW * CH)
    rpt = NP2 // NS
    pltpu.sync_copy(za_hbm.at[pl.ds(s * rpt, rpt)], agg_sp.at[pl.ds(s * rpt, rpt)])
    pltpu.sync_copy(st_hbm, stv)

    descs = []
    for j in range(CPW):
        descs.append(pltpu.async_copy(
            dst_hbm.at[pl.ds(base + j * CH, CH)], dst2d.at[j], isem))
    for d in descs:
        d.wait()

    @pl.when(wid < TAIL_W)
    def _():
        pltpu.sync_copy(dst_hbm.at[pl.ds(TAIL_BASE + wid * CH, CH)],
                        dst2d.at[CPW])

    plsc.subcore_barrier()

    def fire_v(g, b):
        pltpu.async_copy(v_hbm.at[pl.ds((base + g * CH) // 2, CH // 2)],
                         vrs.at[b], svs[b])

    def wait_v(b):
        pltpu.make_async_copy(v_hbm.at[pl.ds(base // 2, CH // 2)], vrs.at[b],
                              svs[b]).wait()

    def wait_s(b):
        pltpu.make_async_copy(mbs.at[b], agg_sp.at[dst2d.at[0]], sss[b]).wait()

    def compute(g, b):
        vr, mb = vrs.at[b], mbs.at[b]

        def row_body(t, _):
            for half in range(2):
                for k in range(4):
                    sl = pl.ds(k * 16, 16)
                    sv = pl.ds(64 * half + k * 16, 16)
                    mb[2 * t + half, sl] = jnp.maximum(
                        vr[t, sv] * stv[0, sl] + stv[1, sl], 0.0)
            return 0

        lax.fori_loop(0, CH // 2, row_body, 0)
        pltpu.async_copy(mb, agg_sp.at[dst2d.at[g]], sss[b], add=True)

    for b in range(NB3 - 1):
        fire_v(b, b)

    def ring_body(j, _):
        for b in range(NB3):
            g = j * NB3 + b
            nxt = g + NB3 - 1

            @pl.when(nxt < CPW)
            def _():
                fire_v(nxt, (b + NB3 - 1) % NB3)

            wait_v(b)

            @pl.when(j > 0)
            def _():
                wait_s(b)

            compute(g, b)
        return 0

    lax.fori_loop(0, CPW // NB3, ring_body, 0)
    for b in range(NB3):
        wait_s(b)

    @pl.when(wid < TAIL_W)
    def _():
        tb = TAIL_BASE + wid * CH
        vr, mb = vrs.at[0], mbs.at[0]
        pltpu.sync_copy(v_hbm.at[pl.ds(tb // 2, CH // 2)], vr)

        def row_body(t, _):
            for half in range(2):
                for k in range(4):
                    sl = pl.ds(k * 16, 16)
                    sv = pl.ds(64 * half + k * 16, 16)
                    mb[2 * t + half, sl] = jnp.maximum(
                        vr[t, sv] * stv[0, sl] + stv[1, sl], 0.0)
            return 0

        lax.fori_loop(0, CH // 2, row_body, 0)
        pltpu.sync_copy(mb, agg_sp.at[dst2d.at[CPW]], add=True)

    plsc.subcore_barrier()

    @pl.when(s == 0)
    def _():
        pltpu.sync_copy(agg_sp, agg_hbm.at[c])


def _sc_pass3(v, dst, st):
    za = jnp.zeros((NP2, HH), jnp.float32)
    mesh = plsc.VectorSubcoreMesh(core_axis_name="c", subcore_axis_name="s")
    f = pl.kernel(
        _sc_pass3_body,
        out_type=jax.ShapeDtypeStruct((NC, NP2, HH), jnp.float32),
        mesh=mesh,
        compiler_params=_SC_PARAMS,
        scratch_types=[
            pltpu.VMEM((CPW + 1, CH), jnp.int32),
            pltpu.VMEM((NB3, CH // 2, 2 * HH), jnp.float32),
            pltpu.VMEM((NB3, CH, HH), jnp.float32),
            pltpu.VMEM((2, HH), jnp.float32),
            pltpu.VMEM_SHARED((NP2, HH), jnp.float32),
            pltpu.SemaphoreType.DMA,
            [pltpu.SemaphoreType.DMA] * NB3,
            [pltpu.SemaphoreType.DMA] * NB3,
        ],
    )
    return f(v, dst, st, za)


def _epi_body(agg_ref, cnt_ref, z0_ref, l1w_ref, l1b_ref, out_ref):
    agg = agg_ref[0, :N] + agg_ref[1, :N]
    cnt = jnp.zeros((1, NP2), jnp.float32)
    for w in range(NW):
        cnt = cnt + cnt_ref[:, pl.ds(w * NP2, NP2)]
    cntc = cnt[:, :N].reshape(N, 1)
    h2 = agg / jnp.maximum(cntc, 1.0)
    out_ref[...] = (z0_ref[...]
                    + jnp.dot(h2, l1w_ref[...], preferred_element_type=jnp.float32)
                    + l1b_ref[...])


def _epilogue(agg2, cnt_flat, z0, l1w, l1b):
    return pl.pallas_call(
        _epi_body,
        out_shape=jax.ShapeDtypeStruct((N, 2), jnp.float32),
    )(agg2, cnt_flat.reshape(1, NW * NP2), z0, l1w, l1b)


def kernel(x, paramsE, paramsS, edge_index):
    pE, pS = paramsE, paramsS
    cat = lambda k: jnp.concatenate([pE[k], pS[k]])[None]
    f1w = jnp.concatenate([pE['f1_w'], pS['f1_w']], axis=1)
    f2w = _blkdiag(pE['f2_w'], pS['f2_w'])
    l0w = _blkdiag(pE['l0_w'], pS['l0_w'])
    l1w = _blkdiag(pE['l1_w'], pS['l1_w'])
    pw = _blkdiag(pE['e1_w'][:H] - pE['e1_w'][H:], pS['e1_w'][:H] - pS['e1_w'][H:])
    qw = _blkdiag(pE['e1_w'][H:], pS['e1_w'][H:])
    e2w = _blkdiag(pE['e2_w'], pS['e2_w'])

    a, bq, z0 = _node_prep(
        x, f1w, cat('f1_b'), cat('f1_g'), cat('f1_be'),
        f2w, cat('f2_b'), cat('f2_g'), cat('f2_be'),
        l0w, cat('l0_b'), pw, qw)

    src = edge_index[0].astype(jnp.int32)
    dst = edge_index[1].astype(jnp.int32)

    u, stats, cnt_flat = _sc_pass1(a, bq, dst, src)
    e2w2 = _blkdiag(e2w, e2w)
    v, st = _tc_pass2(u, stats, cat('e1_g'), cat('e1_be'),
                      e2w, e2w2, cat('e2_b'), cat('e2_g'), cat('e2_be'))
    agg2 = _sc_pass3(v, dst, st)
    return _epilogue(agg2, cnt_flat, z0, l1w, cat('l1_b'))
